# baseline (speedup 1.0000x reference)
import functools

import jax
import jax.numpy as jnp
from jax import lax
from jax.experimental import pallas as pl
from jax.experimental.pallas import tpu as pltpu
from jax.experimental.pallas import tpu_sc as plsc

B = 128
D = 256
NUM_CODES = 2048
VOCAB = 100000
CPW = 8

LANES = 16
PACK = 2
BSLICE = PACK * LANES
CHUNK = 3200
QCHUNK = 400
NQ = CHUNK // QCHUNK
NBLK = QCHUNK // LANES
NPASS = B // BSLICE
TABW = NUM_CODES * LANES
LAST_START = VOCAB - CHUNK


def _mmt_pack_body(w_ref, xlo_ref, xhi_ref, o_ref):
    ylo = lax.dot_general(
        w_ref[...], xlo_ref[...], (((1,), (1,)), ((), ())),
        preferred_element_type=jnp.float32).astype(jnp.bfloat16)
    yhi = lax.dot_general(
        w_ref[...], xhi_ref[...], (((1,), (1,)), ((), ())),
        preferred_element_type=jnp.float32).astype(jnp.bfloat16)
    ulo = lax.bitcast_convert_type(ylo, jnp.uint16).astype(jnp.uint32)
    uhi = lax.bitcast_convert_type(yhi, jnp.uint16).astype(jnp.uint32)
    o_ref[...] = lax.bitcast_convert_type((uhi << 16) | ulo, jnp.int32)[None]


@functools.partial(
    pl.kernel,
    out_type=jax.ShapeDtypeStruct((VOCAB, B), jnp.float32),
    mesh=plsc.VectorSubcoreMesh(
        core_axis_name="c", subcore_axis_name="s", num_cores=2,
        num_subcores=16),
    scratch_types=[
        pltpu.VMEM((CPW * CHUNK,), jnp.int32),
        pltpu.VMEM((TABW,), jnp.int32),
        pltpu.VMEM((TABW,), jnp.int32),
        pltpu.VMEM((QCHUNK, BSLICE), jnp.float32),
        pltpu.VMEM((QCHUNK, BSLICE), jnp.float32),
        pltpu.SemaphoreType.DMA,
        pltpu.SemaphoreType.DMA,
        pltpu.SemaphoreType.DMA,
    ],
    compiler_params=pltpu.CompilerParams(
        needs_layout_passes=False, use_tc_tiling_on_sc=False),
)
def _sc_gather_sum(mkt_hbm, idxt_hbm, out_hbm, idx_v, tab_a, tab_b,
                   out_a, out_b, tab_sem, osem_a, osem_b):
    wid = lax.axis_index("c") * 16 + lax.axis_index("s")
    start = jnp.minimum(wid * CHUNK, LAST_START)
    iota = lax.broadcasted_iota(jnp.int32, (LANES,), 0)

    for j in range(CPW):
        pltpu.sync_copy(idxt_hbm.at[j, pl.ds(start, CHUNK)],
                        idx_v.at[pl.ds(j * CHUNK, CHUNK)])

    @plsc.parallel_loop(0, CPW * CHUNK // LANES)
    def _scale(g):
        idx_v[pl.ds(g * LANES, LANES)] = idx_v[pl.ds(g * LANES, LANES)] << 4

    def tab_start(q, tab_ref):
        pltpu.async_copy(mkt_hbm.at[pl.ds(q * TABW, TABW)], tab_ref, tab_sem)

    def tab_wait(tab_ref):
        pltpu.make_async_copy(mkt_hbm.at[pl.ds(0, TABW)], tab_ref,
                              tab_sem).wait()

    def out_start(q, quarter, out_ref, sem):
        pltpu.async_copy(
            out_ref,
            out_hbm.at[pl.ds(start + quarter * QCHUNK, QCHUNK),
                       pl.ds(q * BSLICE, BSLICE)], sem)

    def out_wait(out_ref, sem):
        pltpu.make_async_copy(
            out_ref, out_hbm.at[pl.ds(0, QCHUNK), pl.ds(0, BSLICE)],
            sem).wait()

    def compute(quarter, tab_ref, out_ref):
        qbase = quarter * QCHUNK

        @plsc.parallel_loop(0, NBLK)
        def _blk(t):
            vbase = qbase + t * LANES
            ivs = [idx_v[pl.ds(j * CHUNK + vbase, LANES)] for j in range(CPW)]
            for l in range(LANES):
                sel = jnp.full((LANES,), l, jnp.int32)
                v = [plsc.load_gather(
                        tab_ref, [jnp.take(ivs[j], sel) + iota])
                     for j in range(CPW)]
                lo = [lax.bitcast_convert_type(vj << 16, jnp.float32)
                      for vj in v]
                hi = [lax.bitcast_convert_type(vj, jnp.float32) for vj in v]
                l01, l23 = lo[0] + lo[1], lo[2] + lo[3]
                l45, l67 = lo[4] + lo[5], lo[6] + lo[7]
                h01, h23 = hi[0] + hi[1], hi[2] + hi[3]
                h45, h67 = hi[4] + hi[5], hi[6] + hi[7]
                out_ref[t * LANES + l, pl.ds(0, LANES)] = (
                    (l01 + l23) + (l45 + l67))
                out_ref[t * LANES + l, pl.ds(LANES, LANES)] = (
                    (h01 + h23) + (h45 + h67))

    tab_start(0, tab_a)

    def pass_body(i4, carry):
        for p_a, tab_ref in ((0, tab_a), (1, tab_b)):
            q = 2 * i4 + p_a
            tab_wait(tab_ref)
            if p_a == 0:
                tab_start(q + 1, tab_b)
            else:
                @pl.when(i4 < NPASS // 2 - 1)
                def _():
                    tab_start(q + 1, tab_a)

            def quarter_body(k2, c2):
                for o_b, out_ref, sem in ((0, out_a, osem_a),
                                          (1, out_b, osem_b)):
                    quarter = 2 * k2 + o_b
                    if p_a == 0:
                        @pl.when((i4 > 0) | (k2 > 0))
                        def _():
                            out_wait(out_ref, sem)
                    else:
                        out_wait(out_ref, sem)
                    compute(quarter, tab_ref, out_ref)
                    out_start(q, quarter, out_ref, sem)
                return c2

            lax.fori_loop(0, NQ // 2, quarter_body, 0, unroll=False)
        return carry

    lax.fori_loop(0, NPASS // 2, pass_body, 0, unroll=False)
    out_wait(out_a, osem_a)
    out_wait(out_b, osem_b)


def kernel(x, W, index_map):
    x4 = x.reshape(NPASS, PACK, LANES, D)
    mk_flat = pl.pallas_call(
        _mmt_pack_body,
        grid=(NPASS,),
        in_specs=[
            pl.BlockSpec((NUM_CODES, D), lambda q: (0, 0)),
            pl.BlockSpec((LANES, D), lambda q: (q, 0)),
            pl.BlockSpec((LANES, D), lambda q: (q, 0)),
        ],
        out_specs=pl.BlockSpec((1, NUM_CODES, LANES), lambda q: (q, 0, 0)),
        out_shape=jax.ShapeDtypeStruct((NPASS, NUM_CODES, LANES), jnp.int32),
    )(W, x4[:, 0].reshape(NPASS * LANES, D), x4[:, 1].reshape(NPASS * LANES, D))
    out_t = _sc_gather_sum(mk_flat.reshape(-1), index_map.T)
    return out_t.T


# speedup vs baseline: 1.0161x; 1.0161x over previous
import functools

import jax
import jax.numpy as jnp
from jax import lax
from jax.experimental import pallas as pl
from jax.experimental.pallas import tpu as pltpu
from jax.experimental.pallas import tpu_sc as plsc

B = 128
D = 256
NUM_CODES = 2048
VOCAB = 100000
CPW = 8

LANES = 16
PACK = 2
BSLICE = PACK * LANES
CHUNK = 3200
QCHUNK = 400
NQ = CHUNK // QCHUNK
NBLK = QCHUNK // LANES
NPASS = B // BSLICE
TABW = NUM_CODES * LANES
LAST_START = VOCAB - CHUNK


def _mmt_body(w_ref, x_ref, o_ref):
    o_ref[...] = lax.dot_general(
        w_ref[...], x_ref[...], (((1,), (1,)), ((), ())),
        preferred_element_type=jnp.float32).astype(jnp.bfloat16)


@functools.partial(
    pl.kernel,
    out_type=jax.ShapeDtypeStruct((VOCAB, B), jnp.float32),
    mesh=plsc.VectorSubcoreMesh(
        core_axis_name="c", subcore_axis_name="s", num_cores=2,
        num_subcores=16),
    scratch_types=[
        pltpu.VMEM((CPW * CHUNK,), jnp.int32),
        pltpu.VMEM((TABW,), jnp.int32),
        pltpu.VMEM((TABW,), jnp.int32),
        pltpu.VMEM((QCHUNK, BSLICE), jnp.float32),
        pltpu.VMEM((QCHUNK, BSLICE), jnp.float32),
        pltpu.SemaphoreType.DMA,
        pltpu.SemaphoreType.DMA,
        pltpu.SemaphoreType.DMA,
    ],
    compiler_params=pltpu.CompilerParams(
        needs_layout_passes=False, use_tc_tiling_on_sc=False),
)
def _sc_gather_sum(mkt_hbm, idxt_hbm, out_hbm, idx_v, tab_a, tab_b,
                   out_a, out_b, tab_sem, osem_a, osem_b):
    wid = lax.axis_index("c") * 16 + lax.axis_index("s")
    start = jnp.minimum(wid * CHUNK, LAST_START)
    iota = lax.broadcasted_iota(jnp.int32, (LANES,), 0)

    for j in range(CPW):
        pltpu.sync_copy(idxt_hbm.at[j, pl.ds(start, CHUNK)],
                        idx_v.at[pl.ds(j * CHUNK, CHUNK)])

    @plsc.parallel_loop(0, CPW * CHUNK // LANES)
    def _scale(g):
        idx_v[pl.ds(g * LANES, LANES)] = idx_v[pl.ds(g * LANES, LANES)] << 4

    def tab_start(q, tab_ref):
        pltpu.async_copy(mkt_hbm.at[pl.ds(q * TABW, TABW)], tab_ref, tab_sem)

    def tab_wait(tab_ref):
        pltpu.make_async_copy(mkt_hbm.at[pl.ds(0, TABW)], tab_ref,
                              tab_sem).wait()

    def out_start(q, quarter, out_ref, sem):
        pltpu.async_copy(
            out_ref,
            out_hbm.at[pl.ds(start + quarter * QCHUNK, QCHUNK),
                       pl.ds(q * BSLICE, BSLICE)], sem)

    def out_wait(out_ref, sem):
        pltpu.make_async_copy(
            out_ref, out_hbm.at[pl.ds(0, QCHUNK), pl.ds(0, BSLICE)],
            sem).wait()

    def compute(quarter, tab_ref, out_ref):
        qbase = quarter * QCHUNK

        @plsc.parallel_loop(0, NBLK)
        def _blk(t):
            vbase = qbase + t * LANES
            ivs = [idx_v[pl.ds(j * CHUNK + vbase, LANES)] for j in range(CPW)]
            for l in range(LANES):
                sel = jnp.full((LANES,), l, jnp.int32)
                v = [plsc.load_gather(
                        tab_ref, [jnp.take(ivs[j], sel) + iota])
                     for j in range(CPW)]
                lo = [lax.bitcast_convert_type(vj << 16, jnp.float32)
                      for vj in v]
                hi = [lax.bitcast_convert_type(vj, jnp.float32) for vj in v]
                l01, l23 = lo[0] + lo[1], lo[2] + lo[3]
                l45, l67 = lo[4] + lo[5], lo[6] + lo[7]
                h01, h23 = hi[0] + hi[1], hi[2] + hi[3]
                h45, h67 = hi[4] + hi[5], hi[6] + hi[7]
                out_ref[t * LANES + l, pl.ds(0, LANES)] = (
                    (l01 + l23) + (l45 + l67))
                out_ref[t * LANES + l, pl.ds(LANES, LANES)] = (
                    (h01 + h23) + (h45 + h67))

    tab_start(0, tab_a)

    def pass_body(i4, carry):
        for p_a, tab_ref in ((0, tab_a), (1, tab_b)):
            q = 2 * i4 + p_a
            tab_wait(tab_ref)
            if p_a == 0:
                tab_start(q + 1, tab_b)
            else:
                @pl.when(i4 < NPASS // 2 - 1)
                def _():
                    tab_start(q + 1, tab_a)

            def quarter_body(k2, c2):
                for o_b, out_ref, sem in ((0, out_a, osem_a),
                                          (1, out_b, osem_b)):
                    quarter = 2 * k2 + o_b
                    if p_a == 0:
                        @pl.when((i4 > 0) | (k2 > 0))
                        def _():
                            out_wait(out_ref, sem)
                    else:
                        out_wait(out_ref, sem)
                    compute(quarter, tab_ref, out_ref)
                    out_start(q, quarter, out_ref, sem)
                return c2

            lax.fori_loop(0, NQ // 2, quarter_body, 0, unroll=False)
        return carry

    lax.fori_loop(0, NPASS // 2, pass_body, 0, unroll=False)
    out_wait(out_a, osem_a)
    out_wait(out_b, osem_b)


def kernel(x, W, index_map):
    perm = jnp.arange(B).reshape(NPASS, PACK, LANES).transpose(0, 2, 1)
    mk_bf = pl.pallas_call(
        _mmt_body,
        out_shape=jax.ShapeDtypeStruct((NUM_CODES, B), jnp.bfloat16),
    )(W, x[perm.reshape(-1)])
    mk_packed = lax.bitcast_convert_type(
        mk_bf.reshape(NUM_CODES, B // PACK, PACK), jnp.int32)
    mk_flat = mk_packed.reshape(NUM_CODES, NPASS, LANES).transpose(
        1, 0, 2).reshape(-1)
    out_t = _sc_gather_sum(mk_flat, index_map.T)
    return out_t.T
